# baseline (speedup 1.0000x reference)
import numpy as np
import jax
import jax.numpy as jnp
from jax import lax
from jax.experimental import pallas as pl
from jax.experimental.pallas import tpu as pltpu
from jax.experimental.pallas import tpu_sc as plsc

N = 10000
E = 160000
G = 100
C = 32
SH = 9
NB = 8
HID = 64
RMAX = 5.0

NC = 2
NS = 16
NW = NC * NS
E2 = 163840
BE = 2048
BN = 1000

_S3 = float(np.sqrt(3.0))
_S5 = float(np.sqrt(5.0))
_S15 = float(np.sqrt(15.0))
_A = float(np.sqrt(2.0 / RMAX))

f32 = jnp.float32


def _sc_gather(table, idx2d, M, D):
    rows = M // 128 // NW
    cap = max(1, (100 * 1024) // (128 * D))
    SB = max(d for d in range(1, rows + 1) if rows % d == 0 and d <= cap)
    nb = rows // SB
    mesh = plsc.VectorSubcoreMesh(core_axis_name="c", subcore_axis_name="s")

    def body(table_hbm, idx_hbm, out_hbm, idx_v, rows_v, sem):
        wid = lax.axis_index("s") * NC + lax.axis_index("c")
        pltpu.sync_copy(idx_hbm.at[pl.ds(wid * rows * 128, rows * 128)], idx_v)

        def step(b, carry):
            pltpu.async_copy(table_hbm.at[idx_v.at[pl.ds(b * SB * 128, SB * 128)]],
                             rows_v, sem).wait()
            pltpu.sync_copy(rows_v,
                            out_hbm.at[pl.ds((wid * rows + b * SB) * 128,
                                             SB * 128)])
            return carry

        lax.fori_loop(0, nb, step, 0)

    fn = pl.kernel(
        body,
        out_type=jax.ShapeDtypeStruct((M, D), f32),
        mesh=mesh,
        compiler_params=pltpu.CompilerParams(use_tc_tiling_on_sc=False),
        scratch_types=[
            pltpu.VMEM((rows * 128,), jnp.int32),
            pltpu.VMEM((SB * 128, D), f32),
            pltpu.SemaphoreType.DMA,
        ],
    )
    return fn(table, idx2d.reshape(M))


def _sc_scatter_add(vals, idx2d, D):
    M = vals.shape[0]
    rows = M // 128 // NW
    slab = N // NS
    mesh = plsc.VectorSubcoreMesh(core_axis_name="c", subcore_axis_name="s")

    cap = max(1, (100 * 1024) // (128 * D))
    SB = max(d for d in range(1, rows + 1) if rows % d == 0 and d <= cap)
    nb = rows // SB

    def body(vals_hbm, idx_hbm, zeros_hbm, out_hbm, idx_v, rows_v, acc):
        cid = lax.axis_index("c")
        sid = lax.axis_index("s")
        wid = sid * NC + cid
        pltpu.sync_copy(zeros_hbm.at[pl.ds(sid * slab, slab)],
                        acc.at[pl.ds(sid * slab, slab)])
        pltpu.sync_copy(idx_hbm.at[pl.ds(wid * rows * 128, rows * 128)], idx_v)
        plsc.subcore_barrier()

        def step(b, carry):
            pltpu.sync_copy(vals_hbm.at[pl.ds((wid * rows + b * SB) * 128,
                                              SB * 128)], rows_v)
            pltpu.sync_copy(rows_v, acc.at[idx_v.at[pl.ds(b * SB * 128,
                                                          SB * 128)]],
                            add=True)
            return carry

        lax.fori_loop(0, nb, step, 0)
        plsc.subcore_barrier()
        pltpu.sync_copy(acc.at[pl.ds(sid * slab, slab)],
                        out_hbm.at[cid, pl.ds(sid * slab, slab)])

    fn = pl.kernel(
        body,
        out_type=jax.ShapeDtypeStruct((2, N, D), f32),
        mesh=mesh,
        compiler_params=pltpu.CompilerParams(use_tc_tiling_on_sc=False),
        scratch_types=[
            pltpu.VMEM((rows * 128,), jnp.int32),
            pltpu.VMEM((SB * 128, D), f32),
            pltpu.VMEM_SHARED((N, D), f32),
        ],
    )
    return fn(vals, idx2d.reshape(M), jnp.zeros((N, D), f32))


def _sc_gather_dyn(table, idx_flat, offrows, D):
    mesh = plsc.VectorSubcoreMesh(core_axis_name="c", subcore_axis_name="s")

    def body(table_hbm, idx_hbm, off_hbm, out_hbm, idx_v, rows_v, sc_v, sem):
        wid = lax.axis_index("s") * NC + lax.axis_index("c")
        pltpu.sync_copy(off_hbm.at[0], sc_v)
        totrows = (sc_v[...][0] + 127) // 128
        nmine = (totrows + NW - 1 - wid) // NW

        def step(k, carry):
            j = wid + k * NW
            pltpu.sync_copy(idx_hbm.at[pl.ds(j * 128, 128)], idx_v)
            pltpu.async_copy(table_hbm.at[idx_v], rows_v, sem).wait()
            pltpu.sync_copy(rows_v, out_hbm.at[pl.ds(j * 128, 128)])
            return carry

        lax.fori_loop(0, nmine, step, 0)

    fn = pl.kernel(
        body,
        out_type=jax.ShapeDtypeStruct((E2, D), f32),
        mesh=mesh,
        compiler_params=pltpu.CompilerParams(use_tc_tiling_on_sc=False),
        scratch_types=[
            pltpu.VMEM((128,), jnp.int32),
            pltpu.VMEM((128, D), f32),
            pltpu.VMEM((16,), jnp.int32),
            pltpu.SemaphoreType.DMA,
        ],
    )
    return fn(table, idx_flat, offrows)


def _sc_scatter_add_dyn(vals, idx_flat, offrows, D, halves=1):
    M = vals.shape[0]
    slab = N // NS
    half_rows = E2 // 128
    mesh = plsc.VectorSubcoreMesh(core_axis_name="c", subcore_axis_name="s")

    def body(vals_hbm, idx_hbm, zeros_hbm, off_hbm, out_hbm,
             idx_v, rows_v, sc_v, acc):
        cid = lax.axis_index("c")
        sid = lax.axis_index("s")
        wid = sid * NC + cid
        pltpu.sync_copy(zeros_hbm.at[pl.ds(sid * slab, slab)],
                        acc.at[pl.ds(sid * slab, slab)])
        pltpu.sync_copy(off_hbm.at[0], sc_v)
        totrows = (sc_v[...][0] + 127) // 128
        nmine = (halves * totrows + NW - 1 - wid) // NW
        plsc.subcore_barrier()

        def step(k, carry):
            vr = wid + k * NW
            j = jnp.where(vr < totrows, vr, half_rows + (vr - totrows))
            pltpu.sync_copy(idx_hbm.at[pl.ds(j * 128, 128)], idx_v)
            pltpu.sync_copy(vals_hbm.at[pl.ds(j * 128, 128)], rows_v)
            pltpu.sync_copy(rows_v, acc.at[idx_v], add=True)
            return carry

        lax.fori_loop(0, nmine, step, 0)
        plsc.subcore_barrier()
        pltpu.sync_copy(acc.at[pl.ds(sid * slab, slab)],
                        out_hbm.at[cid, pl.ds(sid * slab, slab)])

    fn = pl.kernel(
        body,
        out_type=jax.ShapeDtypeStruct((2, N, D), f32),
        mesh=mesh,
        compiler_params=pltpu.CompilerParams(use_tc_tiling_on_sc=False),
        scratch_types=[
            pltpu.VMEM((128,), jnp.int32),
            pltpu.VMEM((128, D), f32),
            pltpu.VMEM((16,), jnp.int32),
            pltpu.VMEM_SHARED((N, D), f32),
        ],
    )
    return fn(vals, idx_flat, jnp.zeros((N, D), f32), offrows)


def _sigmoid(x):
    return 1.0 / (1.0 + jnp.exp(-x))


def _dsilu(x):
    s = _sigmoid(x)
    return s + x * s * (1.0 - s)


def _sh_rows(ux, uy, uz):
    one = jnp.ones_like(ux)
    return [one, _S3 * ux, _S3 * uy, _S3 * uz,
            _S15 * ux * uy, _S15 * uy * uz,
            0.5 * _S5 * (3.0 * uz * uz - 1.0), _S15 * ux * uz,
            0.5 * _S15 * (ux * ux - uy * uy)]


def _radial_rows(ln, want_grad):
    u = ln * (1.0 / RMAX)
    u2 = u * u
    u4 = u2 * u2
    u5 = u4 * u
    u6 = u4 * u2
    u7 = u6 * u
    u8 = u4 * u4
    mask = (u < 1.0).astype(f32)
    env = (1.0 - 28.0 * u6 + 48.0 * u7 - 21.0 * u8) * mask
    rb = ln + 1e-9
    inv_rb = 1.0 / rb
    rrows = []
    drows = []
    if want_grad:
        denv = (-168.0 * u5 + 336.0 * u6 - 168.0 * u7) * (mask * (1.0 / RMAX))
    for n in range(1, NB + 1):
        k = float(n * np.pi / RMAX)
        sn = jnp.sin(k * ln)
        sn_rb = sn * inv_rb
        rrows.append(_A * sn_rb * env)
        if want_grad:
            cs = jnp.cos(k * ln)
            drows.append(_A * (env * (k * cs - sn_rb) * inv_rb + sn_rb * denv))
    Rm = jnp.concatenate(rrows, axis=0)
    if not want_grad:
        return Rm, None
    return Rm, jnp.concatenate(drows, axis=0)


def _edge_common(geo_ref, r_ref, wr1t_ref, br1_ref, wr2t_ref):
    g = geo_ref[...]
    ux, uy, uz, ln = g[0:1], g[1:2], g[2:3], g[3:4]
    ys = _sh_rows(ux, uy, uz)
    Rm = r_ref[...]
    a = jnp.dot(wr1t_ref[...], Rm, preferred_element_type=f32) + br1_ref[...]
    h = a * _sigmoid(a)
    rw = jnp.dot(wr2t_ref[...], h, preferred_element_type=f32)
    return (ux, uy, uz, ln), ys, a, rw


def _prep_call(node_attrs, W_embed, ae_col):
    def body(attrs_ref, we_ref, ae_ref, nf0_ref, e0_ref):
        attrs = attrs_ref[...]
        nf0_ref[...] = jnp.dot(attrs, we_ref[...], preferred_element_type=f32)
        e0_ref[...] = jnp.dot(attrs, ae_ref[...], preferred_element_type=f32)

    return pl.pallas_call(
        body,
        grid=(N // BN,),
        in_specs=[
            pl.BlockSpec((BN, 10), lambda i: (i, 0)),
            pl.BlockSpec((10, C), lambda i: (0, 0)),
            pl.BlockSpec((10, 1), lambda i: (0, 0)),
        ],
        out_specs=[
            pl.BlockSpec((BN, C), lambda i: (i, 0)),
            pl.BlockSpec((BN, 1), lambda i: (i, 0)),
        ],
        out_shape=[
            jax.ShapeDtypeStruct((N, C), f32),
            jax.ShapeDtypeStruct((N, 1), f32),
        ],
    )(node_attrs, W_embed, ae_col)


def _geo_call(posrows, shiftsT, srcdst2):
    def body(p_ref, s_ref, sd_ref, sdl_ref, len_ref):
        d8 = (p_ref[1] - p_ref[0]).T
        v = d8[0:3] + s_ref[0:3]
        ln = jnp.sqrt(jnp.sum(v * v, axis=0, keepdims=True) + 1e-12)
        u = v / ln
        sdf = lax.bitcast_convert_type(sd_ref[...], f32)
        zero = jnp.zeros_like(ln)
        sdl_ref[...] = jnp.concatenate([sdf, u, ln, zero, zero], axis=0).T
        len_ref[...] = ln.reshape(BE // 128, 128)

    return pl.pallas_call(
        body,
        grid=(E2 // BE,),
        in_specs=[
            pl.BlockSpec((2, BE, 8), lambda i: (0, i, 0)),
            pl.BlockSpec((4, BE), lambda i: (0, i)),
            pl.BlockSpec((2, BE), lambda i: (0, i)),
        ],
        out_specs=[
            pl.BlockSpec((BE, 8), lambda i: (i, 0)),
            pl.BlockSpec((BE // 128, 128), lambda i: (i, 0)),
        ],
        out_shape=[
            jax.ShapeDtypeStruct((E2, 8), f32),
            jax.ShapeDtypeStruct((E2 // 128, 128), f32),
        ],
    )(posrows, shiftsT, srcdst2)


TRASH = E2
E2P = E2 + 128


def _cumsum_call(lens2d, triu):
    nsteps = E2 // BE

    def body(len_ref, tri_ref, tgt_ref, tot_ref, run_ref):
        i = pl.program_id(0)

        @pl.when(i == 0)
        def _():
            run_ref[...] = jnp.zeros_like(run_ref)

        mrow = (len_ref[...].reshape(1, BE) < RMAX)
        mf = mrow.astype(f32)
        incl = jnp.dot(mf, tri_ref[...], preferred_element_type=f32)
        base = run_ref[0, 0]
        pos = base + (incl - mf)
        tgt = jnp.where(mrow, pos, float(TRASH))
        tgt_ref[...] = tgt.astype(jnp.int32).reshape(BE // 128, 128)
        newtot = base + jnp.sum(mf)
        run_ref[...] = jnp.broadcast_to(newtot, (1, 1))

        @pl.when(i == nsteps - 1)
        def _():
            tot_ref[...] = jnp.broadcast_to(newtot, (1, 16)).astype(jnp.int32)

    return pl.pallas_call(
        body,
        grid=(nsteps,),
        in_specs=[
            pl.BlockSpec((BE // 128, 128), lambda i: (i, 0)),
            pl.BlockSpec((BE, BE), lambda i: (0, 0)),
        ],
        out_specs=[
            pl.BlockSpec((BE // 128, 128), lambda i: (i, 0)),
            pl.BlockSpec((1, 16), lambda i: (0, 0)),
        ],
        out_shape=[
            jax.ShapeDtypeStruct((E2 // 128, 128), jnp.int32),
            jax.ShapeDtypeStruct((1, 16), jnp.int32),
        ],
        scratch_shapes=[pltpu.VMEM((1, 1), f32)],
    )(lens2d, triu)


def _sc_place(tgt_flat, sdl):
    per_sub = E2 // 128 // NW
    mesh = plsc.VectorSubcoreMesh(core_axis_name="c", subcore_axis_name="s")

    def body(tgt_hbm, sdl_hbm, out_hbm, idx_v, rows_v):
        wid = lax.axis_index("s") * NC + lax.axis_index("c")
        base = wid * per_sub * 128
        pltpu.sync_copy(tgt_hbm.at[pl.ds(base, per_sub * 128)], idx_v)
        pltpu.sync_copy(sdl_hbm.at[pl.ds(base, per_sub * 128)], rows_v)
        pltpu.sync_copy(rows_v, out_hbm.at[idx_v])

    fn = pl.kernel(
        body,
        out_type=jax.ShapeDtypeStruct((E2P, 8), f32),
        mesh=mesh,
        compiler_params=pltpu.CompilerParams(use_tc_tiling_on_sc=False),
        scratch_types=[
            pltpu.VMEM((E2 // 128 // NW * 128,), jnp.int32),
            pltpu.VMEM((E2 // 128 // NW * 128, 8), f32),
        ],
    )
    return fn(tgt_flat, sdl)


BC = 128


def _unpack_call(sdlc, totrow, g):
    def body(s_ref, tot_ref, sd_ref, geo_ref, r_ref, dr_ref):
        i = pl.program_id(0)
        t8 = s_ref[...].T
        gidx = i * BC + lax.broadcasted_iota(jnp.int32, (1, BC), 1)
        valid = gidx < tot_ref[0, 0]
        sd = lax.bitcast_convert_type(t8[0:2], jnp.int32)
        sd_ref[...] = jnp.where(valid, sd, 0)
        u = jnp.where(valid, t8[2:5], 0.0)
        ln = jnp.where(valid, t8[5:6], 100.0)
        u_ln = jnp.concatenate([u, ln], axis=0)
        geo_ref[...] = u_ln
        Rm, dRm = _radial_rows(ln, True)
        r_ref[...] = Rm
        dr_ref[...] = dRm

    return pl.pallas_call(
        body,
        grid=(g,),
        in_specs=[
            pl.BlockSpec((BC, 8), lambda i: (i, 0)),
            pl.BlockSpec((1, 16), lambda i: (0, 0)),
        ],
        out_specs=[
            pl.BlockSpec((2, BC), lambda i: (0, i)),
            pl.BlockSpec((4, BC), lambda i: (0, i)),
            pl.BlockSpec((NB, BC), lambda i: (0, i)),
            pl.BlockSpec((NB, BC), lambda i: (0, i)),
        ],
        out_shape=[
            jax.ShapeDtypeStruct((2, E2), jnp.int32),
            jax.ShapeDtypeStruct((4, E2), f32),
            jax.ShapeDtypeStruct((NB, E2), f32),
            jax.ShapeDtypeStruct((NB, E2), f32),
        ],
    )(sdlc, totrow)


def _edge_fwd_call(geo, Rarr, nfsrc, wr1t, br1c, wr2t, w288T, g):
    def body(geo_ref, r_ref, nf_ref, wr1t_ref, br1_ref, wr2t_ref, wm_ref,
             t_ref):
        _, ys, _, rw = _edge_common(geo_ref, r_ref, wr1t_ref, br1_ref,
                                    wr2t_ref)
        msgT = nf_ref[...].T * rw
        mY = jnp.concatenate([msgT * ys[s] for s in range(SH)], axis=0)
        t_ref[...] = jnp.dot(wm_ref[...], mY,
                             preferred_element_type=f32).T

    return pl.pallas_call(
        body,
        grid=(g,),
        in_specs=[
            pl.BlockSpec((4, BC), lambda i: (0, i)),
            pl.BlockSpec((NB, BC), lambda i: (0, i)),
            pl.BlockSpec((BC, C), lambda i: (i, 0)),
            pl.BlockSpec((HID, NB), lambda i: (0, 0)),
            pl.BlockSpec((HID, 1), lambda i: (0, 0)),
            pl.BlockSpec((C, HID), lambda i: (0, 0)),
            pl.BlockSpec((C, SH * C), lambda i: (0, 0)),
        ],
        out_specs=pl.BlockSpec((BC, C), lambda i: (i, 0)),
        out_shape=jax.ShapeDtypeStruct((E2, C), f32),
    )(geo, Rarr, nfsrc, wr1t, br1c, wr2t, w288T)


def _node_fwd_call(z2, nf, node_attrs, wself, wattr, wread):
    def body(z_ref, nf_ref, at_ref, ws_ref, wa_ref, wr_ref,
             nfo_ref, pre_ref, en_ref):
        pre = (z_ref[0] + z_ref[1]
               + jnp.dot(nf_ref[...], ws_ref[...], preferred_element_type=f32)
               + jnp.dot(at_ref[...], wa_ref[...], preferred_element_type=f32))
        nf2 = pre * _sigmoid(pre)
        nfo_ref[...] = nf2
        pre_ref[...] = pre
        en_ref[...] = jnp.dot(nf2, wr_ref[...], preferred_element_type=f32)

    return pl.pallas_call(
        body,
        grid=(N // BN,),
        in_specs=[
            pl.BlockSpec((2, BN, C), lambda i: (0, i, 0)),
            pl.BlockSpec((BN, C), lambda i: (i, 0)),
            pl.BlockSpec((BN, 10), lambda i: (i, 0)),
            pl.BlockSpec((C, C), lambda i: (0, 0)),
            pl.BlockSpec((10, C), lambda i: (0, 0)),
            pl.BlockSpec((C, 1), lambda i: (0, 0)),
        ],
        out_specs=[
            pl.BlockSpec((BN, C), lambda i: (i, 0)),
            pl.BlockSpec((BN, C), lambda i: (i, 0)),
            pl.BlockSpec((BN, 1), lambda i: (i, 0)),
        ],
        out_shape=[
            jax.ShapeDtypeStruct((N, C), f32),
            jax.ShapeDtypeStruct((N, C), f32),
            jax.ShapeDtypeStruct((N, 1), f32),
        ],
    )(z2, nf, node_attrs, wself, wattr, wread)


def _node_b_call(z2, nf1, node_attrs, enA, wself, wattr, wread,
                 wread0_row, wself1T):
    def body(z_ref, nf_ref, at_ref, ea_ref, ws_ref, wa_ref, wr_ref,
             w0r_ref, ws1t_ref, ni_ref, gpb_ref, gp_ref):
        pre = (z_ref[0] + z_ref[1]
               + jnp.dot(nf_ref[...], ws_ref[...], preferred_element_type=f32)
               + jnp.dot(at_ref[...], wa_ref[...], preferred_element_type=f32))
        nf2 = pre * _sigmoid(pre)
        ni_ref[...] = ea_ref[...] + jnp.dot(nf2, wr_ref[...],
                                            preferred_element_type=f32)
        g_pre = wr_ref[...].T * _dsilu(pre)
        gpb_ref[...] = g_pre
        gp_ref[...] = w0r_ref[...] + jnp.dot(g_pre, ws1t_ref[...],
                                             preferred_element_type=f32)

    return pl.pallas_call(
        body,
        grid=(N // BN,),
        in_specs=[
            pl.BlockSpec((2, BN, C), lambda i: (0, i, 0)),
            pl.BlockSpec((BN, C), lambda i: (i, 0)),
            pl.BlockSpec((BN, 10), lambda i: (i, 0)),
            pl.BlockSpec((BN, 1), lambda i: (i, 0)),
            pl.BlockSpec((C, C), lambda i: (0, 0)),
            pl.BlockSpec((10, C), lambda i: (0, 0)),
            pl.BlockSpec((C, 1), lambda i: (0, 0)),
            pl.BlockSpec((1, C), lambda i: (0, 0)),
            pl.BlockSpec((C, C), lambda i: (0, 0)),
        ],
        out_specs=[
            pl.BlockSpec((BN, 1), lambda i: (i, 0)),
            pl.BlockSpec((BN, C), lambda i: (i, 0)),
            pl.BlockSpec((BN, C), lambda i: (i, 0)),
        ],
        out_shape=[
            jax.ShapeDtypeStruct((N, 1), f32),
            jax.ShapeDtypeStruct((N, C), f32),
            jax.ShapeDtypeStruct((N, C), f32),
        ],
    )(z2, nf1, node_attrs, enA, wself, wattr, wread, wread0_row, wself1T)


def _node_bwda_call(gs2, gnf1p, preA):
    def body(g_ref, gp_ref, pre_ref, out_ref):
        g_nf1 = g_ref[0] + g_ref[1] + gp_ref[...]
        out_ref[...] = g_nf1 * _dsilu(pre_ref[...])

    return pl.pallas_call(
        body,
        grid=(N // BN,),
        in_specs=[
            pl.BlockSpec((2, BN, C), lambda i: (0, i, 0)),
            pl.BlockSpec((BN, C), lambda i: (i, 0)),
            pl.BlockSpec((BN, C), lambda i: (i, 0)),
        ],
        out_specs=pl.BlockSpec((BN, C), lambda i: (i, 0)),
        out_shape=jax.ShapeDtypeStruct((N, C), f32),
    )(gs2, gnf1p, preA)


def _edge_bwd_geom(ys, uxyz, ln, gY, g_len):
    ux, uy, uz = uxyz[0], uxyz[1], uxyz[2]
    gx = _S3 * gY[1] + _S15 * (uy * gY[4] + uz * gY[7] + ux * gY[8])
    gy = _S3 * gY[2] + _S15 * (ux * gY[4] + uz * gY[5] - uy * gY[8])
    gz = _S3 * gY[3] + _S15 * (uy * gY[5] + ux * gY[7]) + 3.0 * _S5 * uz * gY[6]
    udotg = ux * gx + uy * gy + uz * gz
    inv_ln = 1.0 / ln
    gvx = (gx - ux * udotg) * inv_ln + g_len * ux
    gvy = (gy - uy * udotg) * inv_ln + g_len * uy
    gvz = (gz - uz * udotg) * inv_ln + g_len * uz
    return gvx, gvy, gvz


def _edge_bwd_b_call(geo, Rarr, dRarr, nfsrc, gt, wr1t, br1c, wr2t, w288,
                     wr1, wr2, g):
    def body(geo_ref, r_ref, dr_ref, nf_ref, gt_ref, wr1t_ref, br1_ref,
             wr2t_ref, wm_ref, wr1_ref, wr2_ref, gs_ref, gvec_ref):
        uxyz, ys, a, rw = _edge_common(geo_ref, r_ref, wr1t_ref, br1_ref,
                                       wr2t_ref)
        ln = geo_ref[3:4]
        nfT = nf_ref[...].T
        msgT = nfT * rw
        gT = gt_ref[...].T
        P_all = jnp.dot(wm_ref[...], gT, preferred_element_type=f32)
        g_msgT = jnp.zeros_like(gT)
        gY = []
        for s in range(SH):
            P = P_all[s * C:(s + 1) * C]
            g_msgT = g_msgT + P * ys[s]
            gY.append(jnp.sum(P * msgT, axis=0, keepdims=True))
        gs_ref[...] = (g_msgT * rw).T
        g_rwT = g_msgT * nfT
        g_aT = jnp.dot(wr2_ref[...], g_rwT, preferred_element_type=f32) * _dsilu(a)
        g_RT = jnp.dot(wr1_ref[...], g_aT, preferred_element_type=f32)
        g_len = jnp.sum(g_RT * dr_ref[...], axis=0, keepdims=True)
        gvx, gvy, gvz = _edge_bwd_geom(ys, uxyz, ln, gY, g_len)
        gvec_ref[...] = jnp.concatenate(
            [gvx, gvy, gvz, jnp.zeros_like(gvx)], axis=0)

    return pl.pallas_call(
        body,
        grid=(g,),
        in_specs=[
            pl.BlockSpec((4, BC), lambda i: (0, i)),
            pl.BlockSpec((NB, BC), lambda i: (0, i)),
            pl.BlockSpec((NB, BC), lambda i: (0, i)),
            pl.BlockSpec((BC, C), lambda i: (i, 0)),
            pl.BlockSpec((BC, C), lambda i: (i, 0)),
            pl.BlockSpec((HID, NB), lambda i: (0, 0)),
            pl.BlockSpec((HID, 1), lambda i: (0, 0)),
            pl.BlockSpec((C, HID), lambda i: (0, 0)),
            pl.BlockSpec((SH * C, C), lambda i: (0, 0)),
            pl.BlockSpec((NB, HID), lambda i: (0, 0)),
            pl.BlockSpec((HID, C), lambda i: (0, 0)),
        ],
        out_specs=[
            pl.BlockSpec((BC, C), lambda i: (i, 0)),
            pl.BlockSpec((4, BC), lambda i: (0, i)),
        ],
        out_shape=[
            jax.ShapeDtypeStruct((E2, C), f32),
            jax.ShapeDtypeStruct((4, E2), f32),
        ],
    )(geo, Rarr, dRarr, nfsrc, gt, wr1t, br1c, wr2t, w288, wr1, wr2)


def _edge_bwd_a_call(geo, Rarr, dRarr, nfsrc, gt, gvecB, wr1t, br1c, wr2t,
                     w288, wr1, wr2, g):
    def body(geo_ref, r_ref, dr_ref, nf_ref, gt_ref, gvb_ref, wr1t_ref,
             br1_ref, wr2t_ref, wm_ref, wr1_ref, wr2_ref, gv_ref):
        uxyz, ys, a, rw = _edge_common(geo_ref, r_ref, wr1t_ref, br1_ref,
                                       wr2t_ref)
        ln = geo_ref[3:4]
        nfT = nf_ref[...].T
        msgT = nfT * rw
        gT = gt_ref[...].T
        P_all = jnp.dot(wm_ref[...], gT, preferred_element_type=f32)
        g_msgT = jnp.zeros_like(gT)
        gY = []
        for s in range(SH):
            P = P_all[s * C:(s + 1) * C]
            g_msgT = g_msgT + P * ys[s]
            gY.append(jnp.sum(P * msgT, axis=0, keepdims=True))
        g_rwT = g_msgT * nfT
        g_aT = jnp.dot(wr2_ref[...], g_rwT, preferred_element_type=f32) * _dsilu(a)
        g_RT = jnp.dot(wr1_ref[...], g_aT, preferred_element_type=f32)
        g_len = jnp.sum(g_RT * dr_ref[...], axis=0, keepdims=True)
        gvx, gvy, gvz = _edge_bwd_geom(ys, uxyz, ln, gY, g_len)
        gvb = gvb_ref[...]
        gvx = gvx + gvb[0:1]
        gvy = gvy + gvb[1:2]
        gvz = gvz + gvb[2:3]
        zero = jnp.zeros((16 - 3, BC), f32)
        gvT = jnp.concatenate([gvx, gvy, gvz, zero], axis=0).T
        gv_ref[0] = gvT
        gv_ref[1] = -gvT

    return pl.pallas_call(
        body,
        grid=(g,),
        in_specs=[
            pl.BlockSpec((4, BC), lambda i: (0, i)),
            pl.BlockSpec((NB, BC), lambda i: (0, i)),
            pl.BlockSpec((NB, BC), lambda i: (0, i)),
            pl.BlockSpec((BC, C), lambda i: (i, 0)),
            pl.BlockSpec((BC, C), lambda i: (i, 0)),
            pl.BlockSpec((4, BC), lambda i: (0, i)),
            pl.BlockSpec((HID, NB), lambda i: (0, 0)),
            pl.BlockSpec((HID, 1), lambda i: (0, 0)),
            pl.BlockSpec((C, HID), lambda i: (0, 0)),
            pl.BlockSpec((SH * C, C), lambda i: (0, 0)),
            pl.BlockSpec((NB, HID), lambda i: (0, 0)),
            pl.BlockSpec((HID, C), lambda i: (0, 0)),
        ],
        out_specs=pl.BlockSpec((2, BC, 16), lambda i: (0, i, 0)),
        out_shape=jax.ShapeDtypeStruct((2, E2, 16), f32),
    )(geo, Rarr, dRarr, nfsrc, gt, gvecB, wr1t, br1c, wr2t, w288, wr1, wr2)


def _seg_call(e0col, intercol, batchcol):
    nsteps = N // BN

    def body(e0_ref, in_ref, b_ref, tot_ref, int_ref):
        i = pl.program_id(0)

        @pl.when(i == 0)
        def _():
            tot_ref[...] = jnp.zeros_like(tot_ref)
            int_ref[...] = jnp.zeros_like(int_ref)

        oh = (b_ref[...] == lax.broadcasted_iota(jnp.int32, (BN, G), 1)
              ).astype(f32)
        int_ref[...] += jnp.sum(in_ref[...] * oh, axis=0, keepdims=True)
        tot_ref[...] += jnp.sum(e0_ref[...] * oh, axis=0, keepdims=True)

        @pl.when(i == nsteps - 1)
        def _():
            tot_ref[...] += int_ref[...]

    return pl.pallas_call(
        body,
        grid=(nsteps,),
        in_specs=[
            pl.BlockSpec((BN, 1), lambda i: (i, 0)),
            pl.BlockSpec((BN, 1), lambda i: (i, 0)),
            pl.BlockSpec((BN, 1), lambda i: (i, 0)),
        ],
        out_specs=[
            pl.BlockSpec((1, G), lambda i: (0, 0)),
            pl.BlockSpec((1, G), lambda i: (0, 0)),
        ],
        out_shape=[
            jax.ShapeDtypeStruct((1, G), f32),
            jax.ShapeDtypeStruct((1, G), f32),
        ],
    )(e0col, intercol, batchcol)


def _forces_call(F2):
    def body(f_ref, out_ref):
        s = f_ref[0] + f_ref[1]
        out_ref[...] = s[:, 0:3]

    return pl.pallas_call(
        body,
        grid=(N // BN,),
        in_specs=[pl.BlockSpec((2, BN, 16), lambda i: (0, i, 0))],
        out_specs=pl.BlockSpec((BN, 3), lambda i: (i, 0)),
        out_shape=jax.ShapeDtypeStruct((N, 3), f32),
    )(F2)


def kernel(positions, edge_index, shifts, node_attrs, batch, atomic_energies,
           W_embed, Wr1_0, br1_0, Wr2_0, Wmix_0, Wself_0, Wattr_0, Wread_0,
           Wr1_1, br1_1, Wr2_1, Wmix_1, Wself_1, Wattr_1, Wread_1):
    pad = E2 - E
    src_p = jnp.concatenate([edge_index[0], jnp.zeros((pad,), edge_index.dtype)])
    dst_p = jnp.concatenate([edge_index[1], jnp.zeros((pad,), edge_index.dtype)])
    idx_src = src_p.reshape(E2 // 128, 128)
    idx_dst = dst_p.reshape(E2 // 128, 128)
    idx_both = jnp.concatenate([src_p, dst_p]).reshape(2 * E2 // 128, 128)

    shifts_pad = jnp.broadcast_to(jnp.array([[100.0, 0.0, 0.0]], f32), (pad, 3))
    shifts_p = jnp.concatenate([shifts.astype(f32), shifts_pad], axis=0)
    shiftsT = jnp.concatenate([shifts_p.T, jnp.zeros((1, E2), f32)], axis=0)

    pos8 = jnp.concatenate([positions.astype(f32), jnp.zeros((N, 5), f32)], 1)

    def _layer_w(Wr1, br1, Wr2, Wmix):
        w288 = Wmix.reshape(C, SH, C).transpose(1, 0, 2).reshape(SH * C, C)
        return (Wr1.T, br1.reshape(HID, 1), Wr2.T,
                w288.T,
                w288,
                Wr1, Wr2)

    wA = _layer_w(Wr1_0, br1_0, Wr2_0, Wmix_0)
    wB = _layer_w(Wr1_1, br1_1, Wr2_1, Wmix_1)

    posrows = _sc_gather(pos8, idx_both, 2 * E2, 8).reshape(2, E2, 8)
    srcdst2 = jnp.stack([src_p, dst_p])
    sdl, lens2d = _geo_call(posrows, shiftsT, srcdst2)

    triu = jnp.asarray(np.triu(np.ones((BE, BE), np.float32)))
    tgt, totrow = _cumsum_call(lens2d, triu)
    sdlc = _sc_place(tgt.reshape(E2), sdl)
    g = (totrow[0, 0] + 127) // 128
    srcdst_c, geoc, Rc, dRc = _unpack_call(sdlc, totrow, g)
    src_c = srcdst_c[0]
    dst_c = srcdst_c[1]

    nf0, e0col = _prep_call(node_attrs.astype(f32), W_embed,
                            atomic_energies.reshape(10, 1))

    nf0src = _sc_gather_dyn(nf0, src_c, totrow, C)
    tA = _edge_fwd_call(geoc, Rc, nf0src, wA[0], wA[1], wA[2], wA[3], g)
    zA2 = _sc_scatter_add_dyn(tA, dst_c, totrow, C)
    nf1, preA, enA = _node_fwd_call(zA2, nf0, node_attrs, Wself_0, Wattr_0,
                                    Wread_0)

    nf1src = _sc_gather_dyn(nf1, src_c, totrow, C)
    tB = _edge_fwd_call(geoc, Rc, nf1src, wB[0], wB[1], wB[2], wB[3], g)
    zB2 = _sc_scatter_add_dyn(tB, dst_c, totrow, C)
    intercol, g_preB, gnf1p = _node_b_call(
        zB2, nf1, node_attrs, enA, Wself_1, Wattr_1, Wread_1,
        Wread_0.reshape(1, C), Wself_1.T)

    gtB = _sc_gather_dyn(g_preB, dst_c, totrow, C)
    g_s, gvecB = _edge_bwd_b_call(geoc, Rc, dRc, nf1src, gtB, wB[0],
                                  wB[1], wB[2], wB[4], wB[5], wB[6], g)
    gs2 = _sc_scatter_add_dyn(g_s, src_c, totrow, C)
    g_preA = _node_bwda_call(gs2, gnf1p, preA)

    gtA = _sc_gather_dyn(g_preA, dst_c, totrow, C)
    gv = _edge_bwd_a_call(geoc, Rc, dRc, nf0src, gtA, gvecB, wA[0],
                          wA[1], wA[2], wA[4], wA[5], wA[6], g)
    F2 = _sc_scatter_add_dyn(gv.reshape(2 * E2, 16),
                             srcdst_c.reshape(2 * E2), totrow, 16, halves=2)
    forces = _forces_call(F2)

    total, inter = _seg_call(e0col, intercol, batch.reshape(N, 1))
    return total.reshape(G), inter.reshape(G), forces


# speedup vs baseline: 1.1501x; 1.1501x over previous
import numpy as np
import jax
import jax.numpy as jnp
from jax import lax
from jax.experimental import pallas as pl
from jax.experimental.pallas import tpu as pltpu
from jax.experimental.pallas import tpu_sc as plsc

N = 10000
E = 160000
G = 100
C = 32
SH = 9
NB = 8
HID = 64
RMAX = 5.0

NC = 2
NS = 16
NW = NC * NS
E2 = 163840
BE = 2048
BN = 1000

_S3 = float(np.sqrt(3.0))
_S5 = float(np.sqrt(5.0))
_S15 = float(np.sqrt(15.0))
_A = float(np.sqrt(2.0 / RMAX))

f32 = jnp.float32


def _sc_gather(table, idx2d, M, D):
    rows = M // 128 // NW
    cap = max(1, (100 * 1024) // (128 * D))
    SB = max(d for d in range(1, rows + 1) if rows % d == 0 and d <= cap)
    nb = rows // SB
    mesh = plsc.VectorSubcoreMesh(core_axis_name="c", subcore_axis_name="s")

    def body(table_hbm, idx_hbm, out_hbm, idx_v, rows_v, sem):
        wid = lax.axis_index("s") * NC + lax.axis_index("c")
        pltpu.sync_copy(idx_hbm.at[pl.ds(wid * rows * 128, rows * 128)], idx_v)

        def step(b, carry):
            pltpu.async_copy(table_hbm.at[idx_v.at[pl.ds(b * SB * 128, SB * 128)]],
                             rows_v, sem).wait()
            pltpu.sync_copy(rows_v,
                            out_hbm.at[pl.ds((wid * rows + b * SB) * 128,
                                             SB * 128)])
            return carry

        lax.fori_loop(0, nb, step, 0)

    fn = pl.kernel(
        body,
        out_type=jax.ShapeDtypeStruct((M, D), f32),
        mesh=mesh,
        compiler_params=pltpu.CompilerParams(use_tc_tiling_on_sc=False),
        scratch_types=[
            pltpu.VMEM((rows * 128,), jnp.int32),
            pltpu.VMEM((SB * 128, D), f32),
            pltpu.SemaphoreType.DMA,
        ],
    )
    return fn(table, idx2d.reshape(M))


def _sc_scatter_add(vals, idx2d, D):
    M = vals.shape[0]
    rows = M // 128 // NW
    slab = N // NS
    mesh = plsc.VectorSubcoreMesh(core_axis_name="c", subcore_axis_name="s")

    cap = max(1, (100 * 1024) // (128 * D))
    SB = max(d for d in range(1, rows + 1) if rows % d == 0 and d <= cap)
    nb = rows // SB

    def body(vals_hbm, idx_hbm, zeros_hbm, out_hbm, idx_v, rows_v, acc):
        cid = lax.axis_index("c")
        sid = lax.axis_index("s")
        wid = sid * NC + cid
        pltpu.sync_copy(zeros_hbm.at[pl.ds(sid * slab, slab)],
                        acc.at[pl.ds(sid * slab, slab)])
        pltpu.sync_copy(idx_hbm.at[pl.ds(wid * rows * 128, rows * 128)], idx_v)
        plsc.subcore_barrier()

        def step(b, carry):
            pltpu.sync_copy(vals_hbm.at[pl.ds((wid * rows + b * SB) * 128,
                                              SB * 128)], rows_v)
            pltpu.sync_copy(rows_v, acc.at[idx_v.at[pl.ds(b * SB * 128,
                                                          SB * 128)]],
                            add=True)
            return carry

        lax.fori_loop(0, nb, step, 0)
        plsc.subcore_barrier()
        pltpu.sync_copy(acc.at[pl.ds(sid * slab, slab)],
                        out_hbm.at[cid, pl.ds(sid * slab, slab)])

    fn = pl.kernel(
        body,
        out_type=jax.ShapeDtypeStruct((2, N, D), f32),
        mesh=mesh,
        compiler_params=pltpu.CompilerParams(use_tc_tiling_on_sc=False),
        scratch_types=[
            pltpu.VMEM((rows * 128,), jnp.int32),
            pltpu.VMEM((SB * 128, D), f32),
            pltpu.VMEM_SHARED((N, D), f32),
        ],
    )
    return fn(vals, idx2d.reshape(M), jnp.zeros((N, D), f32))


def _sigmoid(x):
    return 1.0 / (1.0 + jnp.exp(-x))


def _dsilu(x):
    s = _sigmoid(x)
    return s + x * s * (1.0 - s)


def _sh_rows(ux, uy, uz):
    one = jnp.ones_like(ux)
    return [one, _S3 * ux, _S3 * uy, _S3 * uz,
            _S15 * ux * uy, _S15 * uy * uz,
            0.5 * _S5 * (3.0 * uz * uz - 1.0), _S15 * ux * uz,
            0.5 * _S15 * (ux * ux - uy * uy)]


def _radial_rows(ln, want_grad):
    u = ln * (1.0 / RMAX)
    u2 = u * u
    u4 = u2 * u2
    u5 = u4 * u
    u6 = u4 * u2
    u7 = u6 * u
    u8 = u4 * u4
    mask = (u < 1.0).astype(f32)
    env = (1.0 - 28.0 * u6 + 48.0 * u7 - 21.0 * u8) * mask
    rb = ln + 1e-9
    inv_rb = 1.0 / rb
    rrows = []
    drows = []
    if want_grad:
        denv = (-168.0 * u5 + 336.0 * u6 - 168.0 * u7) * (mask * (1.0 / RMAX))
    for n in range(1, NB + 1):
        k = float(n * np.pi / RMAX)
        sn = jnp.sin(k * ln)
        sn_rb = sn * inv_rb
        rrows.append(_A * sn_rb * env)
        if want_grad:
            cs = jnp.cos(k * ln)
            drows.append(_A * (env * (k * cs - sn_rb) * inv_rb + sn_rb * denv))
    Rm = jnp.concatenate(rrows, axis=0)
    if not want_grad:
        return Rm, None
    return Rm, jnp.concatenate(drows, axis=0)


def _edge_common(geo_ref, r_ref, wr1t_ref, br1_ref, wr2t_ref):
    g = geo_ref[...]
    ux, uy, uz, ln = g[0:1], g[1:2], g[2:3], g[3:4]
    ys = _sh_rows(ux, uy, uz)
    Rm = r_ref[...]
    a = jnp.dot(wr1t_ref[...], Rm, preferred_element_type=f32) + br1_ref[...]
    h = a * _sigmoid(a)
    rw = jnp.dot(wr2t_ref[...], h, preferred_element_type=f32)
    return (ux, uy, uz, ln), ys, a, rw


def _prep_call(node_attrs, W_embed, ae_col):
    def body(attrs_ref, we_ref, ae_ref, nf0_ref, e0_ref):
        attrs = attrs_ref[...]
        nf0_ref[...] = jnp.dot(attrs, we_ref[...], preferred_element_type=f32)
        e0_ref[...] = jnp.dot(attrs, ae_ref[...], preferred_element_type=f32)

    return pl.pallas_call(
        body,
        grid=(N // BN,),
        in_specs=[
            pl.BlockSpec((BN, 10), lambda i: (i, 0)),
            pl.BlockSpec((10, C), lambda i: (0, 0)),
            pl.BlockSpec((10, 1), lambda i: (0, 0)),
        ],
        out_specs=[
            pl.BlockSpec((BN, C), lambda i: (i, 0)),
            pl.BlockSpec((BN, 1), lambda i: (i, 0)),
        ],
        out_shape=[
            jax.ShapeDtypeStruct((N, C), f32),
            jax.ShapeDtypeStruct((N, 1), f32),
        ],
    )(node_attrs, W_embed, ae_col)


def _geo_call(posrows, shiftsT):
    def body(p_ref, s_ref, geo_ref, r_ref, dr_ref):
        d8 = (p_ref[1] - p_ref[0]).T
        v = d8[0:3] + s_ref[0:3]
        ln = jnp.sqrt(jnp.sum(v * v, axis=0, keepdims=True) + 1e-12)
        u = v / ln
        geo_ref[...] = jnp.concatenate([u, ln], axis=0)
        Rm, dRm = _radial_rows(ln, True)
        r_ref[...] = Rm
        dr_ref[...] = dRm

    return pl.pallas_call(
        body,
        grid=(E2 // BE,),
        in_specs=[
            pl.BlockSpec((2, BE, 8), lambda i: (0, i, 0)),
            pl.BlockSpec((4, BE), lambda i: (0, i)),
        ],
        out_specs=[
            pl.BlockSpec((4, BE), lambda i: (0, i)),
            pl.BlockSpec((NB, BE), lambda i: (0, i)),
            pl.BlockSpec((NB, BE), lambda i: (0, i)),
        ],
        out_shape=[
            jax.ShapeDtypeStruct((4, E2), f32),
            jax.ShapeDtypeStruct((NB, E2), f32),
            jax.ShapeDtypeStruct((NB, E2), f32),
        ],
    )(posrows, shiftsT)


def _edge_fwd_call(geo, Rarr, nfsrc, wr1t, br1c, wr2t, w288T):
    def body(geo_ref, r_ref, nf_ref, wr1t_ref, br1_ref, wr2t_ref, wm_ref,
             t_ref):
        _, ys, _, rw = _edge_common(geo_ref, r_ref, wr1t_ref, br1_ref,
                                    wr2t_ref)
        msgT = nf_ref[...].T * rw
        mY = jnp.concatenate([msgT * ys[s] for s in range(SH)], axis=0)
        t_ref[...] = jnp.dot(wm_ref[...], mY,
                             preferred_element_type=f32).T

    return pl.pallas_call(
        body,
        grid=(E2 // BE,),
        in_specs=[
            pl.BlockSpec((4, BE), lambda i: (0, i)),
            pl.BlockSpec((NB, BE), lambda i: (0, i)),
            pl.BlockSpec((BE, C), lambda i: (i, 0)),
            pl.BlockSpec((HID, NB), lambda i: (0, 0)),
            pl.BlockSpec((HID, 1), lambda i: (0, 0)),
            pl.BlockSpec((C, HID), lambda i: (0, 0)),
            pl.BlockSpec((C, SH * C), lambda i: (0, 0)),
        ],
        out_specs=pl.BlockSpec((BE, C), lambda i: (i, 0)),
        out_shape=jax.ShapeDtypeStruct((E2, C), f32),
    )(geo, Rarr, nfsrc, wr1t, br1c, wr2t, w288T)


def _node_fwd_call(z2, nf, node_attrs, wself, wattr, wread):
    def body(z_ref, nf_ref, at_ref, ws_ref, wa_ref, wr_ref,
             nfo_ref, pre_ref, en_ref):
        pre = (z_ref[0] + z_ref[1]
               + jnp.dot(nf_ref[...], ws_ref[...], preferred_element_type=f32)
               + jnp.dot(at_ref[...], wa_ref[...], preferred_element_type=f32))
        nf2 = pre * _sigmoid(pre)
        nfo_ref[...] = nf2
        pre_ref[...] = pre
        en_ref[...] = jnp.dot(nf2, wr_ref[...], preferred_element_type=f32)

    return pl.pallas_call(
        body,
        grid=(N // BN,),
        in_specs=[
            pl.BlockSpec((2, BN, C), lambda i: (0, i, 0)),
            pl.BlockSpec((BN, C), lambda i: (i, 0)),
            pl.BlockSpec((BN, 10), lambda i: (i, 0)),
            pl.BlockSpec((C, C), lambda i: (0, 0)),
            pl.BlockSpec((10, C), lambda i: (0, 0)),
            pl.BlockSpec((C, 1), lambda i: (0, 0)),
        ],
        out_specs=[
            pl.BlockSpec((BN, C), lambda i: (i, 0)),
            pl.BlockSpec((BN, C), lambda i: (i, 0)),
            pl.BlockSpec((BN, 1), lambda i: (i, 0)),
        ],
        out_shape=[
            jax.ShapeDtypeStruct((N, C), f32),
            jax.ShapeDtypeStruct((N, C), f32),
            jax.ShapeDtypeStruct((N, 1), f32),
        ],
    )(z2, nf, node_attrs, wself, wattr, wread)


def _node_b_call(z2, nf1, node_attrs, enA, wself, wattr, wread,
                 wread0_row, wself1T):
    def body(z_ref, nf_ref, at_ref, ea_ref, ws_ref, wa_ref, wr_ref,
             w0r_ref, ws1t_ref, ni_ref, gpb_ref, gp_ref):
        pre = (z_ref[0] + z_ref[1]
               + jnp.dot(nf_ref[...], ws_ref[...], preferred_element_type=f32)
               + jnp.dot(at_ref[...], wa_ref[...], preferred_element_type=f32))
        nf2 = pre * _sigmoid(pre)
        ni_ref[...] = ea_ref[...] + jnp.dot(nf2, wr_ref[...],
                                            preferred_element_type=f32)
        g_pre = wr_ref[...].T * _dsilu(pre)
        gpb_ref[...] = g_pre
        gp_ref[...] = w0r_ref[...] + jnp.dot(g_pre, ws1t_ref[...],
                                             preferred_element_type=f32)

    return pl.pallas_call(
        body,
        grid=(N // BN,),
        in_specs=[
            pl.BlockSpec((2, BN, C), lambda i: (0, i, 0)),
            pl.BlockSpec((BN, C), lambda i: (i, 0)),
            pl.BlockSpec((BN, 10), lambda i: (i, 0)),
            pl.BlockSpec((BN, 1), lambda i: (i, 0)),
            pl.BlockSpec((C, C), lambda i: (0, 0)),
            pl.BlockSpec((10, C), lambda i: (0, 0)),
            pl.BlockSpec((C, 1), lambda i: (0, 0)),
            pl.BlockSpec((1, C), lambda i: (0, 0)),
            pl.BlockSpec((C, C), lambda i: (0, 0)),
        ],
        out_specs=[
            pl.BlockSpec((BN, 1), lambda i: (i, 0)),
            pl.BlockSpec((BN, C), lambda i: (i, 0)),
            pl.BlockSpec((BN, C), lambda i: (i, 0)),
        ],
        out_shape=[
            jax.ShapeDtypeStruct((N, 1), f32),
            jax.ShapeDtypeStruct((N, C), f32),
            jax.ShapeDtypeStruct((N, C), f32),
        ],
    )(z2, nf1, node_attrs, enA, wself, wattr, wread, wread0_row, wself1T)


def _node_bwda_call(gs2, gnf1p, preA):
    def body(g_ref, gp_ref, pre_ref, out_ref):
        g_nf1 = g_ref[0] + g_ref[1] + gp_ref[...]
        out_ref[...] = g_nf1 * _dsilu(pre_ref[...])

    return pl.pallas_call(
        body,
        grid=(N // BN,),
        in_specs=[
            pl.BlockSpec((2, BN, C), lambda i: (0, i, 0)),
            pl.BlockSpec((BN, C), lambda i: (i, 0)),
            pl.BlockSpec((BN, C), lambda i: (i, 0)),
        ],
        out_specs=pl.BlockSpec((BN, C), lambda i: (i, 0)),
        out_shape=jax.ShapeDtypeStruct((N, C), f32),
    )(gs2, gnf1p, preA)


def _edge_bwd_geom(ys, uxyz, ln, gY, g_len):
    ux, uy, uz = uxyz[0], uxyz[1], uxyz[2]
    gx = _S3 * gY[1] + _S15 * (uy * gY[4] + uz * gY[7] + ux * gY[8])
    gy = _S3 * gY[2] + _S15 * (ux * gY[4] + uz * gY[5] - uy * gY[8])
    gz = _S3 * gY[3] + _S15 * (uy * gY[5] + ux * gY[7]) + 3.0 * _S5 * uz * gY[6]
    udotg = ux * gx + uy * gy + uz * gz
    inv_ln = 1.0 / ln
    gvx = (gx - ux * udotg) * inv_ln + g_len * ux
    gvy = (gy - uy * udotg) * inv_ln + g_len * uy
    gvz = (gz - uz * udotg) * inv_ln + g_len * uz
    return gvx, gvy, gvz


def _edge_bwd_b_call(geo, Rarr, dRarr, nfsrc, gt, wr1t, br1c, wr2t, w288,
                     wr1, wr2):
    def body(geo_ref, r_ref, dr_ref, nf_ref, gt_ref, wr1t_ref, br1_ref,
             wr2t_ref, wm_ref, wr1_ref, wr2_ref, gs_ref, gvec_ref):
        uxyz, ys, a, rw = _edge_common(geo_ref, r_ref, wr1t_ref, br1_ref,
                                       wr2t_ref)
        ln = geo_ref[3:4]
        nfT = nf_ref[...].T
        msgT = nfT * rw
        gT = gt_ref[...].T
        P_all = jnp.dot(wm_ref[...], gT, preferred_element_type=f32)
        g_msgT = jnp.zeros_like(gT)
        gY = []
        for s in range(SH):
            P = P_all[s * C:(s + 1) * C]
            g_msgT = g_msgT + P * ys[s]
            gY.append(jnp.sum(P * msgT, axis=0, keepdims=True))
        gs_ref[...] = (g_msgT * rw).T
        g_rwT = g_msgT * nfT
        g_aT = jnp.dot(wr2_ref[...], g_rwT, preferred_element_type=f32) * _dsilu(a)
        g_RT = jnp.dot(wr1_ref[...], g_aT, preferred_element_type=f32)
        g_len = jnp.sum(g_RT * dr_ref[...], axis=0, keepdims=True)
        gvx, gvy, gvz = _edge_bwd_geom(ys, uxyz, ln, gY, g_len)
        gvec_ref[...] = jnp.concatenate(
            [gvx, gvy, gvz, jnp.zeros_like(gvx)], axis=0)

    return pl.pallas_call(
        body,
        grid=(E2 // BE,),
        in_specs=[
            pl.BlockSpec((4, BE), lambda i: (0, i)),
            pl.BlockSpec((NB, BE), lambda i: (0, i)),
            pl.BlockSpec((NB, BE), lambda i: (0, i)),
            pl.BlockSpec((BE, C), lambda i: (i, 0)),
            pl.BlockSpec((BE, C), lambda i: (i, 0)),
            pl.BlockSpec((HID, NB), lambda i: (0, 0)),
            pl.BlockSpec((HID, 1), lambda i: (0, 0)),
            pl.BlockSpec((C, HID), lambda i: (0, 0)),
            pl.BlockSpec((SH * C, C), lambda i: (0, 0)),
            pl.BlockSpec((NB, HID), lambda i: (0, 0)),
            pl.BlockSpec((HID, C), lambda i: (0, 0)),
        ],
        out_specs=[
            pl.BlockSpec((BE, C), lambda i: (i, 0)),
            pl.BlockSpec((4, BE), lambda i: (0, i)),
        ],
        out_shape=[
            jax.ShapeDtypeStruct((E2, C), f32),
            jax.ShapeDtypeStruct((4, E2), f32),
        ],
    )(geo, Rarr, dRarr, nfsrc, gt, wr1t, br1c, wr2t, w288, wr1, wr2)


def _edge_bwd_a_call(geo, Rarr, dRarr, nfsrc, gt, gvecB, wr1t, br1c, wr2t,
                     w288, wr1, wr2):
    def body(geo_ref, r_ref, dr_ref, nf_ref, gt_ref, gvb_ref, wr1t_ref,
             br1_ref, wr2t_ref, wm_ref, wr1_ref, wr2_ref, gv_ref):
        uxyz, ys, a, rw = _edge_common(geo_ref, r_ref, wr1t_ref, br1_ref,
                                       wr2t_ref)
        ln = geo_ref[3:4]
        nfT = nf_ref[...].T
        msgT = nfT * rw
        gT = gt_ref[...].T
        P_all = jnp.dot(wm_ref[...], gT, preferred_element_type=f32)
        g_msgT = jnp.zeros_like(gT)
        gY = []
        for s in range(SH):
            P = P_all[s * C:(s + 1) * C]
            g_msgT = g_msgT + P * ys[s]
            gY.append(jnp.sum(P * msgT, axis=0, keepdims=True))
        g_rwT = g_msgT * nfT
        g_aT = jnp.dot(wr2_ref[...], g_rwT, preferred_element_type=f32) * _dsilu(a)
        g_RT = jnp.dot(wr1_ref[...], g_aT, preferred_element_type=f32)
        g_len = jnp.sum(g_RT * dr_ref[...], axis=0, keepdims=True)
        gvx, gvy, gvz = _edge_bwd_geom(ys, uxyz, ln, gY, g_len)
        gvb = gvb_ref[...]
        gvx = gvx + gvb[0:1]
        gvy = gvy + gvb[1:2]
        gvz = gvz + gvb[2:3]
        zero = jnp.zeros((16 - 3, BE), f32)
        gvT = jnp.concatenate([gvx, gvy, gvz, zero], axis=0).T
        gv_ref[0] = gvT
        gv_ref[1] = -gvT

    return pl.pallas_call(
        body,
        grid=(E2 // BE,),
        in_specs=[
            pl.BlockSpec((4, BE), lambda i: (0, i)),
            pl.BlockSpec((NB, BE), lambda i: (0, i)),
            pl.BlockSpec((NB, BE), lambda i: (0, i)),
            pl.BlockSpec((BE, C), lambda i: (i, 0)),
            pl.BlockSpec((BE, C), lambda i: (i, 0)),
            pl.BlockSpec((4, BE), lambda i: (0, i)),
            pl.BlockSpec((HID, NB), lambda i: (0, 0)),
            pl.BlockSpec((HID, 1), lambda i: (0, 0)),
            pl.BlockSpec((C, HID), lambda i: (0, 0)),
            pl.BlockSpec((SH * C, C), lambda i: (0, 0)),
            pl.BlockSpec((NB, HID), lambda i: (0, 0)),
            pl.BlockSpec((HID, C), lambda i: (0, 0)),
        ],
        out_specs=pl.BlockSpec((2, BE, 16), lambda i: (0, i, 0)),
        out_shape=jax.ShapeDtypeStruct((2, E2, 16), f32),
    )(geo, Rarr, dRarr, nfsrc, gt, gvecB, wr1t, br1c, wr2t, w288, wr1, wr2)


def _seg_call(e0col, intercol, batchcol):
    nsteps = N // BN

    def body(e0_ref, in_ref, b_ref, tot_ref, int_ref):
        i = pl.program_id(0)

        @pl.when(i == 0)
        def _():
            tot_ref[...] = jnp.zeros_like(tot_ref)
            int_ref[...] = jnp.zeros_like(int_ref)

        oh = (b_ref[...] == lax.broadcasted_iota(jnp.int32, (BN, G), 1)
              ).astype(f32)
        int_ref[...] += jnp.sum(in_ref[...] * oh, axis=0, keepdims=True)
        tot_ref[...] += jnp.sum(e0_ref[...] * oh, axis=0, keepdims=True)

        @pl.when(i == nsteps - 1)
        def _():
            tot_ref[...] += int_ref[...]

    return pl.pallas_call(
        body,
        grid=(nsteps,),
        in_specs=[
            pl.BlockSpec((BN, 1), lambda i: (i, 0)),
            pl.BlockSpec((BN, 1), lambda i: (i, 0)),
            pl.BlockSpec((BN, 1), lambda i: (i, 0)),
        ],
        out_specs=[
            pl.BlockSpec((1, G), lambda i: (0, 0)),
            pl.BlockSpec((1, G), lambda i: (0, 0)),
        ],
        out_shape=[
            jax.ShapeDtypeStruct((1, G), f32),
            jax.ShapeDtypeStruct((1, G), f32),
        ],
    )(e0col, intercol, batchcol)


def _forces_call(F2):
    def body(f_ref, out_ref):
        s = f_ref[0] + f_ref[1]
        out_ref[...] = s[:, 0:3]

    return pl.pallas_call(
        body,
        grid=(N // BN,),
        in_specs=[pl.BlockSpec((2, BN, 16), lambda i: (0, i, 0))],
        out_specs=pl.BlockSpec((BN, 3), lambda i: (i, 0)),
        out_shape=jax.ShapeDtypeStruct((N, 3), f32),
    )(F2)


def kernel(positions, edge_index, shifts, node_attrs, batch, atomic_energies,
           W_embed, Wr1_0, br1_0, Wr2_0, Wmix_0, Wself_0, Wattr_0, Wread_0,
           Wr1_1, br1_1, Wr2_1, Wmix_1, Wself_1, Wattr_1, Wread_1):
    pad = E2 - E
    src_p = jnp.concatenate([edge_index[0], jnp.zeros((pad,), edge_index.dtype)])
    dst_p = jnp.concatenate([edge_index[1], jnp.zeros((pad,), edge_index.dtype)])
    idx_src = src_p.reshape(E2 // 128, 128)
    idx_dst = dst_p.reshape(E2 // 128, 128)
    idx_both = jnp.concatenate([src_p, dst_p]).reshape(2 * E2 // 128, 128)

    shifts_pad = jnp.broadcast_to(jnp.array([[100.0, 0.0, 0.0]], f32), (pad, 3))
    shifts_p = jnp.concatenate([shifts.astype(f32), shifts_pad], axis=0)
    shiftsT = jnp.concatenate([shifts_p.T, jnp.zeros((1, E2), f32)], axis=0)

    pos8 = jnp.concatenate([positions.astype(f32), jnp.zeros((N, 5), f32)], 1)

    def _layer_w(Wr1, br1, Wr2, Wmix):
        w288 = Wmix.reshape(C, SH, C).transpose(1, 0, 2).reshape(SH * C, C)
        return (Wr1.T, br1.reshape(HID, 1), Wr2.T,
                w288.T,
                w288,
                Wr1, Wr2)

    wA = _layer_w(Wr1_0, br1_0, Wr2_0, Wmix_0)
    wB = _layer_w(Wr1_1, br1_1, Wr2_1, Wmix_1)

    posrows = _sc_gather(pos8, idx_both, 2 * E2, 8).reshape(2, E2, 8)
    geo, Rarr, dRarr = _geo_call(posrows, shiftsT)

    nf0, e0col = _prep_call(node_attrs.astype(f32), W_embed,
                            atomic_energies.reshape(10, 1))

    nf0src = _sc_gather(nf0, idx_src, E2, C)
    tA = _edge_fwd_call(geo, Rarr, nf0src, wA[0], wA[1], wA[2], wA[3])
    zA2 = _sc_scatter_add(tA, idx_dst, C)
    nf1, preA, enA = _node_fwd_call(zA2, nf0, node_attrs, Wself_0, Wattr_0,
                                    Wread_0)

    nf1src = _sc_gather(nf1, idx_src, E2, C)
    tB = _edge_fwd_call(geo, Rarr, nf1src, wB[0], wB[1], wB[2], wB[3])
    zB2 = _sc_scatter_add(tB, idx_dst, C)
    intercol, g_preB, gnf1p = _node_b_call(
        zB2, nf1, node_attrs, enA, Wself_1, Wattr_1, Wread_1,
        Wread_0.reshape(1, C), Wself_1.T)

    gtB = _sc_gather(g_preB, idx_dst, E2, C)
    g_s, gvecB = _edge_bwd_b_call(geo, Rarr, dRarr, nf1src, gtB, wB[0],
                                  wB[1], wB[2], wB[4], wB[5], wB[6])
    gs2 = _sc_scatter_add(g_s, idx_src, C)
    g_preA = _node_bwda_call(gs2, gnf1p, preA)

    gtA = _sc_gather(g_preA, idx_dst, E2, C)
    gv = _edge_bwd_a_call(geo, Rarr, dRarr, nf0src, gtA, gvecB, wA[0],
                          wA[1], wA[2], wA[4], wA[5], wA[6])
    F2 = _sc_scatter_add(gv.reshape(2 * E2, 16), idx_both, 16)
    forces = _forces_call(F2)

    total, inter = _seg_call(e0col, intercol, batch.reshape(N, 1))
    return total.reshape(G), inter.reshape(G), forces
